# Initial kernel scaffold; baseline (speedup 1.0000x reference)
import jax
import jax.numpy as jnp
from jax.experimental import pallas as pl


def kernel(x, thresholds, table, W_pred, b_pred, dy1, dx1, c1, dy2, dx2, c2):
    raise NotImplementedError("write your pallas kernel here")


# speedup vs baseline: 1.0494x; 1.0494x over previous
import jax
import jax.numpy as jnp
from jax.experimental import pallas as pl
from jax.experimental.pallas import tpu as pltpu

M = 8
K = 10
L = 5
C = 3
H = 64
W = 64
N = 32
DOUT = 64
NCLS = 10
NWORDS = 2 ** K
PAD = L // 2


def _mm_body(a_ref, b_ref, o_ref):
    o_ref[...] = jnp.dot(a_ref[...], b_ref[...],
                         preferred_element_type=jnp.float32)


def _final_matmul(flat, W_pred):
    return pl.pallas_call(
        _mm_body,
        out_shape=jax.ShapeDtypeStruct((N, NCLS), jnp.float32),
    )(flat, W_pred)


def kernel(x, thresholds, table, W_pred, b_pred, dy1, dx1, c1, dy2, dx2, c2):
    xp = jnp.pad(x, ((0, 0), (0, 0), (PAD, PAD), (PAD, PAD)))

    def _slice_one(c, dy, dx):
        patch = jax.lax.dynamic_slice(
            xp, (jnp.int32(0), c.astype(jnp.int32), dy.astype(jnp.int32),
                 dx.astype(jnp.int32)), (N, 1, H, W))
        return patch[:, 0]

    def _pair(c1k, dy1k, dx1k, c2k, dy2k, dx2k):
        return _slice_one(c1k, dy1k, dx1k) - _slice_one(c2k, dy2k, dx2k)

    diffs = jax.vmap(jax.vmap(_pair))(c1, dy1, dx1, c2, dy2, dx2)
    z = diffs - thresholds[:, :, None, None, None]
    bits = z > 0.0
    pw = (2 ** jnp.arange(K, dtype=jnp.int32))[None, :, None, None, None]
    words = jnp.sum(bits.astype(jnp.int32) * pw, axis=1)
    soft = jax.nn.sigmoid(z)
    conf = jnp.prod(jnp.where(bits, soft, 1.0 - soft), axis=1)
    out = jnp.zeros((N, H, W, DOUT), dtype=jnp.float32)
    for m in range(M):
        idx = words[m] + m * NWORDS
        out = out + jnp.take(table, idx, axis=0) * conf[m][..., None]
    out = jnp.transpose(out, (0, 3, 1, 2))
    pooled = out.reshape(N, DOUT, H // 2, 2, W // 2, 2).mean(axis=(3, 5))
    flat = pooled.reshape(N, -1)
    return _final_matmul(flat, W_pred) + b_pred


# speedup vs baseline: 10.2737x; 9.7900x over previous
import functools

import jax
import jax.numpy as jnp
from jax import lax
from jax.experimental import pallas as pl
from jax.experimental.pallas import tpu as pltpu
from jax.experimental.pallas import tpu_sc as plsc

M = 8
K = 10
L = 5
C = 3
H = 64
W = 64
N = 32
DOUT = 64
NCLS = 10
NWORDS = 2 ** K
PAD = L // 2
HW = H * W
HP = H // 2
WP = W // 2


def _stage1_body(off_ref, thr_ref, xp_ref, idx_ref, conf_ref):
    m = pl.program_id(0)
    word = jnp.zeros((N, H, W), jnp.int32)
    conf = jnp.full((N, H, W), 0.25, jnp.float32)
    for k in range(K):
        c1k = off_ref[m, k, 0]
        dy1k = off_ref[m, k, 1]
        dx1k = off_ref[m, k, 2]
        c2k = off_ref[m, k, 3]
        dy2k = off_ref[m, k, 4]
        dx2k = off_ref[m, k, 5]
        v1 = xp_ref[:, c1k, pl.ds(dy1k, H), :]
        v2 = xp_ref[:, c2k, pl.ds(dy2k, H), :]
        p1 = pltpu.roll(v1, 68 - dx1k, axis=2)[:, :, :W]
        p2 = pltpu.roll(v2, 68 - dx2k, axis=2)[:, :, :W]
        z = (p1 - p2) - thr_ref[m, k]
        bit = z > 0.0
        word = word + jnp.where(bit, jnp.int32(1 << k), jnp.int32(0))
        s = 1.0 / (1.0 + jnp.exp(-z))
        conf = conf * jnp.where(bit, s, 1.0 - s)
    idx_ref[0] = (word + m * NWORDS).reshape(N, HW)
    conf_ref[0] = conf.reshape(N, HW)


def _stage1(xp, offs, thr):
    return pl.pallas_call(
        _stage1_body,
        grid=(M,),
        in_specs=[
            pl.BlockSpec(memory_space=pltpu.SMEM),
            pl.BlockSpec(memory_space=pltpu.SMEM),
            pl.BlockSpec((N, C, H + 2 * PAD, W + 2 * PAD),
                         lambda m: (0, 0, 0, 0)),
        ],
        out_specs=[
            pl.BlockSpec((1, N, HW), lambda m: (m, 0, 0)),
            pl.BlockSpec((1, N, HW), lambda m: (m, 0, 0)),
        ],
        out_shape=[
            jax.ShapeDtypeStruct((M, N, HW), jnp.int32),
            jax.ShapeDtypeStruct((M, N, HW), jnp.float32),
        ],
    )(offs, thr, xp)


def _sc_body(idx_hbm, conf_hbm, table_hbm, out_hbm,
             idx_v, conf_v, gbuf, acc, gsem, lsem):
    cid = lax.axis_index("c")
    sid = lax.axis_index("s")
    n = sid * 2 + cid

    for m in range(M):
        pltpu.async_copy(idx_hbm.at[m, n], idx_v.at[m], lsem)
        pltpu.async_copy(conf_hbm.at[m, n], conf_v.at[m], lsem)
    for m in range(M):
        pltpu.make_async_copy(idx_hbm.at[m, n], idx_v.at[m], lsem).wait()
        pltpu.make_async_copy(conf_hbm.at[m, n], conf_v.at[m], lsem).wait()

    @pl.loop(0, 4 * HP)
    def _chunk(c):
        h = c // 2
        wh = lax.rem(c, 2)
        i = c // 4
        hh = lax.rem(h, 2)
        px0 = h * W + wh * 32
        for m in range(M):
            pltpu.async_copy(
                table_hbm.at[idx_v.at[m, pl.ds(px0, 32)]], gbuf.at[m], gsem)
        for m in range(M):
            pltpu.make_async_copy(
                table_hbm.at[idx_v.at[m, pl.ds(px0, 32)]], gbuf.at[m],
                gsem).wait()
        jbase = wh * 16
        cvecs = [[conf_v[m, pl.ds(px0 + 16 * half, 16)] for half in range(2)]
                 for m in range(M)]
        for p in range(16):
            a = [None] * 4
            for m in range(M):
                for b in range(2):
                    lane = 2 * p + b
                    cv = jnp.full((16,), cvecs[m][lane // 16][lane % 16],
                                  jnp.float32)
                    for q in range(4):
                        r = gbuf[m, 2 * p + b, pl.ds(16 * q, 16)]
                        t = cv * r
                        a[q] = t if a[q] is None else a[q] + t

            @pl.when(hh == 0)
            def _(a=a, p=p):
                for q in range(4):
                    acc[jbase + p, pl.ds(16 * q, 16)] = a[q]

            @pl.when(hh == 1)
            def _(a=a, p=p):
                for q in range(4):
                    plsc.addupdate(acc.at[jbase + p, pl.ds(16 * q, 16)], a[q])

        @pl.when(hh == 1)
        def _():
            pltpu.sync_copy(acc.at[pl.ds(jbase, 16)],
                            out_hbm.at[n, i, pl.ds(jbase, 16)])


@functools.cache
def _sc_gather():
    mesh = plsc.VectorSubcoreMesh(core_axis_name="c", subcore_axis_name="s")
    return pl.kernel(
        _sc_body,
        out_type=jax.ShapeDtypeStruct((N, HP, WP, DOUT), jnp.float32),
        mesh=mesh,
        scratch_types=[
            pltpu.VMEM((M, HW), jnp.int32),
            pltpu.VMEM((M, HW), jnp.float32),
            pltpu.VMEM((M, 32, DOUT), jnp.float32),
            pltpu.VMEM((WP, DOUT), jnp.float32),
            pltpu.SemaphoreType.DMA,
            pltpu.SemaphoreType.DMA,
        ],
        compiler_params=pltpu.CompilerParams(use_tc_tiling_on_sc=False),
    )


def _mm_body(a_ref, bt_ref, o_ref):
    o_ref[...] = jax.lax.dot_general(
        a_ref[...], bt_ref[...], (((1,), (1,)), ((), ())),
        preferred_element_type=jnp.float32)


def _stage3(flat, wt_t):
    return pl.pallas_call(
        _mm_body,
        out_shape=jax.ShapeDtypeStruct((N, NCLS), jnp.float32),
    )(flat, wt_t)


def kernel(x, thresholds, table, W_pred, b_pred, dy1, dx1, c1, dy2, dx2, c2):
    xp = jnp.pad(x, ((0, 0), (0, 0), (PAD, PAD), (PAD, PAD)))
    offs = jnp.stack([c1, dy1, dx1, c2, dy2, dx2], axis=-1).astype(jnp.int32)
    idx, conf = _stage1(xp, offs, thresholds)
    pooled = _sc_gather()(idx, conf, table)
    flat = pooled.reshape(N, HP * WP * DOUT)
    wt_t = W_pred.reshape(DOUT, HP * WP, NCLS).transpose(2, 1, 0).reshape(
        NCLS, HP * WP * DOUT)
    return _stage3(flat, wt_t) + b_pred
